# baseline (speedup 1.0000x reference)
import functools

import jax
import jax.numpy as jnp
from jax import lax
from jax.experimental import pallas as pl
from jax.experimental.pallas import tpu as pltpu
from jax.experimental.pallas import tpu_sc as plsc

WORD_DIM = 1024
N_ROWS = 4 * 8192
NUM_CORES = 2
NUM_SUBCORES = 16
NUM_WORKERS = NUM_CORES * NUM_SUBCORES
ROWS_PER_WORKER = N_ROWS // NUM_WORKERS
CHUNK = 32
NUM_CHUNKS = ROWS_PER_WORKER // CHUNK
NBUF = 3


@jax.jit
def _gather_sc(idx_flat, table):
    mesh = plsc.VectorSubcoreMesh(
        core_axis_name="c",
        subcore_axis_name="s",
        num_cores=NUM_CORES,
        num_subcores=NUM_SUBCORES,
    )

    @functools.partial(
        pl.kernel,
        mesh=mesh,
        out_type=jax.ShapeDtypeStruct((N_ROWS, WORD_DIM), jnp.float32),
        scratch_types=[
            pltpu.VMEM((NUM_CHUNKS, CHUNK), jnp.int32),
            pltpu.VMEM((NBUF, CHUNK, WORD_DIM), jnp.float32),
            [pltpu.SemaphoreType.DMA] * NBUF,
            [pltpu.SemaphoreType.DMA] * NBUF,
        ],
    )
    def body(idx_hbm, table_hbm, out_hbm, idx_v, rows_v, gsem, ssem):
        wid = lax.axis_index("s") * NUM_CORES + lax.axis_index("c")
        base = wid * ROWS_PER_WORKER
        pltpu.sync_copy(idx_hbm.at[wid], idx_v)

        def gather(c):
            return pltpu.async_copy(
                table_hbm.at[idx_v.at[c]], rows_v.at[c % NBUF], gsem[c % NBUF]
            )

        def scatter(c):
            return pltpu.async_copy(
                rows_v.at[c % NBUF],
                out_hbm.at[pl.ds(base + c * CHUNK, CHUNK)],
                ssem[c % NBUF],
            )

        gathers = [None] * NUM_CHUNKS
        scatters = [None] * NUM_CHUNKS
        for c in range(NBUF - 1):
            gathers[c] = gather(c)
        for c in range(NUM_CHUNKS):
            if c + NBUF - 1 < NUM_CHUNKS:
                if c >= 1:
                    scatters[c - 1].wait()
                gathers[c + NBUF - 1] = gather(c + NBUF - 1)
            gathers[c].wait()
            scatters[c] = scatter(c)
        for c in range(max(0, NUM_CHUNKS - NBUF), NUM_CHUNKS):
            scatters[c].wait()

    return body(idx_flat, table)


def kernel(pos_idxs, position_enc_weight):
    idx = pos_idxs.reshape(NUM_WORKERS, NUM_CHUNKS, CHUNK).astype(jnp.int32)
    out = _gather_sc(idx, position_enc_weight)
    return out.reshape(pos_idxs.shape + (WORD_DIM,))


# speedup vs baseline: 1.0006x; 1.0006x over previous
import functools

import jax
import jax.numpy as jnp
from jax import lax
from jax.experimental import pallas as pl
from jax.experimental.pallas import tpu as pltpu
from jax.experimental.pallas import tpu_sc as plsc

WORD_DIM = 1024
N_ROWS = 4 * 8192
NUM_CORES = 2
NUM_SUBCORES = 16
NUM_WORKERS = NUM_CORES * NUM_SUBCORES
ROWS_PER_WORKER = N_ROWS // NUM_WORKERS
CHUNK = 16
NUM_CHUNKS = ROWS_PER_WORKER // CHUNK
NBUF = 6


@jax.jit
def _gather_sc(idx_flat, table):
    mesh = plsc.VectorSubcoreMesh(
        core_axis_name="c",
        subcore_axis_name="s",
        num_cores=NUM_CORES,
        num_subcores=NUM_SUBCORES,
    )

    @functools.partial(
        pl.kernel,
        mesh=mesh,
        out_type=jax.ShapeDtypeStruct((N_ROWS, WORD_DIM), jnp.float32),
        scratch_types=[
            pltpu.VMEM((NUM_CHUNKS, CHUNK), jnp.int32),
            pltpu.VMEM((NBUF, CHUNK, WORD_DIM), jnp.float32),
            [pltpu.SemaphoreType.DMA] * NBUF,
            [pltpu.SemaphoreType.DMA] * NBUF,
        ],
    )
    def body(idx_hbm, table_hbm, out_hbm, idx_v, rows_v, gsem, ssem):
        wid = lax.axis_index("s") * NUM_CORES + lax.axis_index("c")
        base = wid * ROWS_PER_WORKER
        pltpu.sync_copy(idx_hbm.at[wid], idx_v)

        def gather(c):
            return pltpu.async_copy(
                table_hbm.at[idx_v.at[c]], rows_v.at[c % NBUF], gsem[c % NBUF]
            )

        def scatter(c):
            return pltpu.async_copy(
                rows_v.at[c % NBUF],
                out_hbm.at[pl.ds(base + c * CHUNK, CHUNK)],
                ssem[c % NBUF],
            )

        gathers = [None] * NUM_CHUNKS
        scatters = [None] * NUM_CHUNKS
        for c in range(NBUF - 1):
            gathers[c] = gather(c)
        for c in range(NUM_CHUNKS):
            if c + NBUF - 1 < NUM_CHUNKS:
                if c >= 1:
                    scatters[c - 1].wait()
                gathers[c + NBUF - 1] = gather(c + NBUF - 1)
            gathers[c].wait()
            scatters[c] = scatter(c)
        for c in range(max(0, NUM_CHUNKS - NBUF), NUM_CHUNKS):
            scatters[c].wait()

    return body(idx_flat, table)


def kernel(pos_idxs, position_enc_weight):
    idx = pos_idxs.reshape(NUM_WORKERS, NUM_CHUNKS, CHUNK).astype(jnp.int32)
    out = _gather_sc(idx, position_enc_weight)
    return out.reshape(pos_idxs.shape + (WORD_DIM,))


# speedup vs baseline: 1.0007x; 1.0001x over previous
import functools

import jax
import jax.numpy as jnp
from jax import lax
from jax.experimental import pallas as pl
from jax.experimental.pallas import tpu as pltpu
from jax.experimental.pallas import tpu_sc as plsc

WORD_DIM = 1024
N_ROWS = 4 * 8192
NUM_CORES = 2
NUM_SUBCORES = 16
NUM_WORKERS = NUM_CORES * NUM_SUBCORES
ROWS_PER_WORKER = N_ROWS // NUM_WORKERS
CHUNK = 32
NUM_CHUNKS = ROWS_PER_WORKER // CHUNK
NBUF = 3


@jax.jit
def _gather_sc(idx_flat, table):
    mesh = plsc.VectorSubcoreMesh(
        core_axis_name="c",
        subcore_axis_name="s",
        num_cores=NUM_CORES,
        num_subcores=NUM_SUBCORES,
    )

    @functools.partial(
        pl.kernel,
        mesh=mesh,
        out_type=jax.ShapeDtypeStruct((N_ROWS, WORD_DIM), jnp.float32),
        scratch_types=[
            pltpu.VMEM((NUM_CHUNKS, CHUNK), jnp.int32),
            pltpu.VMEM((NBUF, CHUNK, WORD_DIM), jnp.float32),
            [pltpu.SemaphoreType.DMA] * NBUF,
            [pltpu.SemaphoreType.DMA] * NBUF,
        ],
    )
    def body(idx_hbm, table_hbm, out_hbm, idx_v, rows_v, gsem, ssem):
        wid = lax.axis_index("s") * NUM_CORES + lax.axis_index("c")
        base = wid * ROWS_PER_WORKER
        pltpu.sync_copy(idx_hbm.at[wid], idx_v)

        def gather(c):
            return pltpu.async_copy(
                table_hbm.at[idx_v.at[c]], rows_v.at[c % NBUF], gsem[c % NBUF]
            )

        def scatter(c):
            return pltpu.async_copy(
                rows_v.at[c % NBUF],
                out_hbm.at[pl.ds(base + c * CHUNK, CHUNK)],
                ssem[c % NBUF],
            )

        gathers = [None] * NUM_CHUNKS
        scatters = [None] * NUM_CHUNKS
        for c in range(NBUF - 1):
            gathers[c] = gather(c)
        for c in range(NUM_CHUNKS):
            if c + NBUF - 1 < NUM_CHUNKS:
                if c >= 1:
                    scatters[c - 1].wait()
                gathers[c + NBUF - 1] = gather(c + NBUF - 1)
            gathers[c].wait()
            scatters[c] = scatter(c)
        for c in range(max(0, NUM_CHUNKS - NBUF), NUM_CHUNKS):
            scatters[c].wait()

    return body(idx_flat, table)


def kernel(pos_idxs, position_enc_weight):
    idx = pos_idxs.reshape(NUM_WORKERS, NUM_CHUNKS, CHUNK).astype(jnp.int32)
    out = _gather_sc(idx, position_enc_weight)
    return out.reshape(pos_idxs.shape + (WORD_DIM,))
